# baseline (speedup 1.0000x reference)
import jax
import jax.numpy as jnp
from jax.experimental import pallas as pl

DEPTH = 2


def _mlp(x, W1, b1, W2, b2):
    return jax.nn.gelu(x @ W1 + b1) @ W2 + b2


def _gno(x, ei, ea, i, Wk1, bk1, Wk2, bk2, Wr, br, Wout, bout):
    src = ei[0]
    dst = ei[1]
    n = x.shape[0]
    deg = jax.ops.segment_sum(jnp.ones((ei.shape[1],), x.dtype), dst, num_segments=n)
    deg = jnp.clip(deg, 1.0)[:, None]
    for l in range(DEPTH):
        k = jax.nn.gelu(ea @ Wk1[i, l] + bk1[i, l]) @ Wk2[i, l] + bk2[i, l]
        msg = jnp.take(x, src, axis=0) * k
        agg = jax.ops.segment_sum(msg, dst, num_segments=n) / deg
        x = jax.nn.gelu(x @ Wr[i, l] + br[i, l] + agg)
    return x @ Wout[i] + bout[i]


def _decode_body(x_ref, w1_ref, b1_ref, w2_ref, b2_ref, o_ref):
    h = jax.nn.gelu(x_ref[...] @ w1_ref[...] + b1_ref[...])
    o_ref[...] = h @ w2_ref[...] + b2_ref[...]


def kernel(nodes, grid, edge_index_11, edge_index_12, edge_index_22, edge_index_23, edge_index_33, edge_index_32, edge_index_21, edge_attr_11, edge_attr_12, edge_attr_22, edge_attr_23, edge_attr_33, edge_attr_32, edge_attr_21, batch_size, image_size, Wp1, bp1, Wp2, bp2, Wd1, bd1, Wd2, bd2, Wk1, bk1, Wk2, bk2, Wr, br, Wout, bout):
    x = jnp.concatenate([nodes, grid], axis=-1)
    x = _mlp(x, Wp1, bp1, Wp2, bp2)
    x = jax.nn.gelu(x)
    P = (Wk1, bk1, Wk2, bk2, Wr, br, Wout, bout)
    n11 = _gno(x, edge_index_11, edge_attr_11, 0, *P)
    n12 = _gno(x, edge_index_12, edge_attr_12, 3, *P)
    n22 = _gno(n12, edge_index_22, edge_attr_22, 1, *P)
    n23 = _gno(n12, edge_index_23, edge_attr_23, 4, *P)
    n33 = _gno(n23, edge_index_33, edge_attr_33, 2, *P)
    n32 = _gno(n33, edge_index_32, edge_attr_32, 5, *P)
    n21 = _gno(n32 + n22, edge_index_21, edge_attr_21, 6, *P)
    x = n21 + n11
    N = x.shape[0]
    B = 2000
    out = pl.pallas_call(
        _decode_body,
        grid=(N // B,),
        in_specs=[
            pl.BlockSpec((B, 64), lambda i: (i, 0)),
            pl.BlockSpec((64, 32), lambda i: (0, 0)),
            pl.BlockSpec((32,), lambda i: (0,)),
            pl.BlockSpec((32, 1), lambda i: (0, 0)),
            pl.BlockSpec((1,), lambda i: (0,)),
        ],
        out_specs=pl.BlockSpec((B, 1), lambda i: (i, 0)),
        out_shape=jax.ShapeDtypeStruct((N, 1), jnp.float32),
    )(x, Wd1, bd1, Wd2, bd2)
    return out


# speedup vs baseline: 2.4280x; 2.4280x over previous
import functools

import jax
import jax.numpy as jnp
from jax import lax
from jax.experimental import pallas as pl
from jax.experimental.pallas import tpu as pltpu
from jax.experimental.pallas import tpu_sc as plsc

N = 50000
NP = 51200
RPT = NP // 16
H = 32
F = 64
E = 800000
EP = 802816
GROUPS = EP // 128
GPT = GROUPS // 16
SCH = 2
NIT = GPT // SCH
DEPTH = 2

_f32 = jnp.float32


@functools.cache
def _mesh():
    return plsc.VectorSubcoreMesh(core_axis_name="c", subcore_axis_name="s")


def _edge_pass_body(x2, k2, src2, dst2, flag16, aggL, aggR,
                    agg_s, xg, kg, srcv, dstv, fv, gsem, ssem):
    c = lax.axis_index("c")
    s = lax.axis_index("s")
    zeros = jnp.zeros((16,), _f32)
    ones = jnp.ones((16,), _f32)

    pltpu.sync_copy(flag16, fv)
    is_deg = jnp.max(fv[...]) > 0

    @pl.loop(0, SCH * 128, unroll=8)
    def _(r):
        xg[r, pl.ds(0, 16)] = zeros
        xg[r, pl.ds(16, 16)] = zeros

    _CROWS = SCH * 128
    for j in range(RPT // _CROWS):
        pltpu.sync_copy(xg, agg_s.at[pl.ds(s * RPT + j * _CROWS, _CROWS)])
    _remz = RPT % _CROWS
    if _remz:
        pltpu.sync_copy(xg.at[pl.ds(0, _remz)],
                        agg_s.at[pl.ds(s * RPT + (RPT // _CROWS) * _CROWS, _remz)])
    plsc.subcore_barrier()

    @pl.when(is_deg)
    def _():
        @pl.loop(0, SCH * 128, unroll=8)
        def _(r):
            xg[r, pl.ds(0, 16)] = ones
            xg[r, pl.ds(16, 16)] = ones

    g0 = s * GPT

    @pl.loop(0, NIT)
    def _(i):
        grow = g0 + i * SCH
        pltpu.sync_copy(dst2.at[pl.ds(grow, SCH)], dstv)

        @pl.when(jnp.logical_not(is_deg))
        def _():
            pltpu.sync_copy(src2.at[pl.ds(grow, SCH)], srcv)
            kd = pltpu.async_copy(k2.at[c].at[pl.ds(grow * 128, SCH * 128)],
                                  kg, gsem)
            gds = [pltpu.async_copy(x2.at[c].at[srcv.at[g]],
                                    xg.at[pl.ds(g * 128, 128)], gsem)
                   for g in range(SCH)]
            kd.wait()
            for gd in gds:
                gd.wait()

            @pl.loop(0, SCH * 128, unroll=8)
            def _(r):
                xg[r, pl.ds(0, 16)] = xg[r, pl.ds(0, 16)] * kg[r, pl.ds(0, 16)]
                xg[r, pl.ds(16, 16)] = xg[r, pl.ds(16, 16)] * kg[r, pl.ds(16, 16)]

        sds = [pltpu.async_copy(xg.at[pl.ds(g * 128, 128)],
                                agg_s.at[dstv.at[g]], ssem, add=True)
               for g in range(SCH)]
        for sd in sds:
            sd.wait()

    plsc.subcore_barrier()

    @pl.when(c == 0)
    def _():
        pltpu.sync_copy(agg_s.at[pl.ds(s * RPT, RPT)], aggL.at[pl.ds(s * RPT, RPT)])

    @pl.when(c == 1)
    def _():
        pltpu.sync_copy(agg_s.at[pl.ds(s * RPT, RPT)], aggR.at[pl.ds(s * RPT, RPT)])


@functools.cache
def _edge_pass_fn():
    return pl.kernel(
        _edge_pass_body,
        out_type=[jax.ShapeDtypeStruct((NP, H), _f32),
                  jax.ShapeDtypeStruct((NP, H), _f32)],
        mesh=_mesh(),
        compiler_params=pltpu.CompilerParams(use_tc_tiling_on_sc=False,
                                            needs_layout_passes=False),
        scratch_types=[
            pltpu.VMEM_SHARED((NP, H), _f32),
            pltpu.VMEM((SCH * 128, H), _f32),
            pltpu.VMEM((SCH * 128, H), _f32),
            pltpu.VMEM((SCH, 128), jnp.int32),
            pltpu.VMEM((SCH, 128), jnp.int32),
            pltpu.VMEM((16,), jnp.int32),
            pltpu.SemaphoreType.DMA,
            pltpu.SemaphoreType.DMA,
        ],
    )


def _edge_pass(x2, k2, src2, dst2, deg_mode=False):
    flag = jnp.full((16,), 1 if deg_mode else 0, jnp.int32)
    return _edge_pass_fn()(x2, k2, src2, dst2, flag)


_NB_TC = 2000
_NGRID = N // _NB_TC
_EB = 8192
_EGRID = EP // _EB


def _encoder_body(xin_ref, w1_ref, b1_ref, w2_ref, b2_ref, o_ref):
    h = jax.nn.gelu(jnp.dot(xin_ref[...], w1_ref[...],
                            preferred_element_type=_f32) + b1_ref[...])
    x = jax.nn.gelu(jnp.dot(h, w2_ref[...],
                            preferred_element_type=_f32) + b2_ref[...])
    o_ref[0] = x[:, :H]
    o_ref[1] = x[:, H:]


def _encoder(x_in, Wp1, bp1, Wp2, bp2):
    return pl.pallas_call(
        _encoder_body,
        grid=(_NGRID,),
        in_specs=[
            pl.BlockSpec((_NB_TC, 12), lambda i: (i, 0)),
            pl.BlockSpec((12, H), lambda i: (0, 0)),
            pl.BlockSpec((H,), lambda i: (0,)),
            pl.BlockSpec((H, F), lambda i: (0, 0)),
            pl.BlockSpec((F,), lambda i: (0,)),
        ],
        out_specs=pl.BlockSpec((2, _NB_TC, H), lambda i: (0, i, 0)),
        out_shape=jax.ShapeDtypeStruct((2, NP, H), _f32),
    )(x_in, Wp1, bp1, Wp2, bp2)


def _edge_mlp_body(ea_ref, w1_ref, b1_ref, w2_ref, b2_ref, k0_ref, k1_ref):
    e = ea_ref[...]
    for l, out in ((0, k0_ref), (1, k1_ref)):
        h = jax.nn.gelu(jnp.dot(e, w1_ref[l], preferred_element_type=_f32)
                        + b1_ref[l])
        k = jnp.dot(h, w2_ref[l], preferred_element_type=_f32) + b2_ref[l]
        out[0] = k[:, :H]
        out[1] = k[:, H:]


def _edge_mlp(ea_p, Wk1i, bk1i, Wk2i, bk2i):
    return pl.pallas_call(
        _edge_mlp_body,
        grid=(_EGRID,),
        in_specs=[
            pl.BlockSpec((_EB, 8), lambda i: (i, 0)),
            pl.BlockSpec((2, 8, F), lambda i: (0, 0, 0)),
            pl.BlockSpec((2, F), lambda i: (0, 0)),
            pl.BlockSpec((2, F, F), lambda i: (0, 0, 0)),
            pl.BlockSpec((2, F), lambda i: (0, 0)),
        ],
        out_specs=[pl.BlockSpec((2, _EB, H), lambda i: (0, i, 0)),
                   pl.BlockSpec((2, _EB, H), lambda i: (0, i, 0))],
        out_shape=[jax.ShapeDtypeStruct((2, EP, H), _f32),
                   jax.ShapeDtypeStruct((2, EP, H), _f32)],
    )(ea_p, Wk1i, bk1i, Wk2i, bk2i)


def _node_update_body(x2_ref, aL_ref, aR_ref, deg_ref, wr_ref, br_ref, o_ref):
    x = jnp.concatenate([x2_ref[0], x2_ref[1]], axis=-1)
    agg = jnp.concatenate([aL_ref[...], aR_ref[...]], axis=-1)
    rdeg = 1.0 / jnp.maximum(deg_ref[...][:, 0:1], 1.0)
    y = jax.nn.gelu(jnp.dot(x, wr_ref[...], preferred_element_type=_f32)
                    + br_ref[...] + agg * rdeg)
    o_ref[0] = y[:, :H]
    o_ref[1] = y[:, H:]


def _node_update(x2, aggL, aggR, deg, Wr_il, br_il):
    return pl.pallas_call(
        _node_update_body,
        grid=(_NGRID,),
        in_specs=[
            pl.BlockSpec((2, _NB_TC, H), lambda i: (0, i, 0)),
            pl.BlockSpec((_NB_TC, H), lambda i: (i, 0)),
            pl.BlockSpec((_NB_TC, H), lambda i: (i, 0)),
            pl.BlockSpec((_NB_TC, H), lambda i: (i, 0)),
            pl.BlockSpec((F, F), lambda i: (0, 0)),
            pl.BlockSpec((F,), lambda i: (0,)),
        ],
        out_specs=pl.BlockSpec((2, _NB_TC, H), lambda i: (0, i, 0)),
        out_shape=jax.ShapeDtypeStruct((2, NP, H), _f32),
    )(x2, aggL, aggR, deg, Wr_il, br_il)


def _block_out_body(x2_ref, w_ref, b_ref, o_ref):
    x = jnp.concatenate([x2_ref[0], x2_ref[1]], axis=-1)
    y = jnp.dot(x, w_ref[...], preferred_element_type=_f32) + b_ref[...]
    o_ref[0] = y[:, :H]
    o_ref[1] = y[:, H:]


def _block_out_res_body(x2_ref, w_ref, b_ref, r2_ref, o_ref):
    x = jnp.concatenate([x2_ref[0], x2_ref[1]], axis=-1)
    y = jnp.dot(x, w_ref[...], preferred_element_type=_f32) + b_ref[...]
    o_ref[0] = y[:, :H] + r2_ref[0]
    o_ref[1] = y[:, H:] + r2_ref[1]


def _block_out(x2, Wout_i, bout_i, res2=None):
    spec2 = pl.BlockSpec((2, _NB_TC, H), lambda i: (0, i, 0))
    in_specs = [spec2,
                pl.BlockSpec((F, F), lambda i: (0, 0)),
                pl.BlockSpec((F,), lambda i: (0,))]
    args = [x2, Wout_i, bout_i]
    body = _block_out_body
    if res2 is not None:
        in_specs.append(spec2)
        args.append(res2)
        body = _block_out_res_body
    return pl.pallas_call(
        body,
        grid=(_NGRID,),
        in_specs=in_specs,
        out_specs=spec2,
        out_shape=jax.ShapeDtypeStruct((2, NP, H), _f32),
    )(*args)


def _decoder_body(a2_ref, b2_ref, w1_ref, b1_ref, w2_ref, b2b_ref, o_ref):
    x = jnp.concatenate([a2_ref[0] + b2_ref[0], a2_ref[1] + b2_ref[1]], axis=-1)
    h = jax.nn.gelu(jnp.dot(x, w1_ref[...], preferred_element_type=_f32)
                    + b1_ref[...])
    o_ref[...] = jnp.dot(h, w2_ref[...], preferred_element_type=_f32) + b2b_ref[...]


def _decoder(a2, b2, Wd1, bd1, Wd2, bd2):
    spec2 = pl.BlockSpec((2, _NB_TC, H), lambda i: (0, i, 0))
    return pl.pallas_call(
        _decoder_body,
        grid=(_NGRID,),
        in_specs=[
            spec2, spec2,
            pl.BlockSpec((F, H), lambda i: (0, 0)),
            pl.BlockSpec((H,), lambda i: (0,)),
            pl.BlockSpec((H, 1), lambda i: (0, 0)),
            pl.BlockSpec((1,), lambda i: (0,)),
        ],
        out_specs=pl.BlockSpec((_NB_TC, 1), lambda i: (i, 0)),
        out_shape=jax.ShapeDtypeStruct((N, 1), _f32),
    )(a2, b2, Wd1, bd1, Wd2, bd2)


def _prep_edges(ei, ea):
    src = jnp.concatenate([ei[0], jnp.zeros((EP - E,), ei.dtype)])
    dst = jnp.concatenate([ei[1], jnp.full((EP - E,), N, ei.dtype)])
    ea_p = jnp.pad(ea, ((0, EP - E), (0, 3)))
    return (src.astype(jnp.int32).reshape(GROUPS, 128),
            dst.astype(jnp.int32).reshape(GROUPS, 128), ea_p)


def kernel(nodes, grid, edge_index_11, edge_index_12, edge_index_22, edge_index_23, edge_index_33, edge_index_32, edge_index_21, edge_attr_11, edge_attr_12, edge_attr_22, edge_attr_23, edge_attr_33, edge_attr_32, edge_attr_21, batch_size, image_size, Wp1, bp1, Wp2, bp2, Wd1, bd1, Wd2, bd2, Wk1, bk1, Wk2, bk2, Wr, br, Wout, bout):
    eis = [edge_index_11, edge_index_12, edge_index_22, edge_index_23,
           edge_index_33, edge_index_32, edge_index_21]
    eas = [edge_attr_11, edge_attr_12, edge_attr_22, edge_attr_23,
           edge_attr_33, edge_attr_32, edge_attr_21]
    prepped = [_prep_edges(ei, ea) for ei, ea in zip(eis, eas)]
    srcs = [p[0] for p in prepped]
    dsts = [p[1] for p in prepped]
    eaps = [p[2] for p in prepped]

    Wk1p = jnp.pad(Wk1, ((0, 0), (0, 0), (0, 3), (0, 0)))

    x_in = jnp.concatenate([nodes, grid], axis=-1)
    x2 = _encoder(x_in, Wp1, bp1, Wp2, bp2)

    deg_cache = {}

    def gno(x2_in, b_idx, i, res2=None):
        k0, k1 = _edge_mlp(eaps[b_idx], Wk1p[i], bk1[i], Wk2[i], bk2[i])
        if b_idx not in deg_cache:
            deg_cache[b_idx], _ = _edge_pass(x2_in, k0, srcs[b_idx],
                                             dsts[b_idx], deg_mode=True)
        deg = deg_cache[b_idx]
        h2 = x2_in
        for l, k2 in ((0, k0), (1, k1)):
            aggL, aggR = _edge_pass(h2, k2, srcs[b_idx], dsts[b_idx])
            h2 = _node_update(h2, aggL, aggR, deg, Wr[i, l], br[i, l])
        return _block_out(h2, Wout[i], bout[i], res2)

    n11 = gno(x2, 0, 0)
    n12 = gno(x2, 1, 3)
    n22 = gno(n12, 2, 1)
    n23 = gno(n12, 3, 4)
    n33 = gno(n23, 4, 2)
    n32p = gno(n33, 5, 5, res2=n22)
    n21 = gno(n32p, 6, 6)

    return _decoder(n21, n11, Wd1, bd1, Wd2, bd2)


# speedup vs baseline: 2.8282x; 1.1649x over previous
import functools

import jax
import jax.numpy as jnp
from jax import lax
from jax.experimental import pallas as pl
from jax.experimental.pallas import tpu as pltpu
from jax.experimental.pallas import tpu_sc as plsc

N = 50000
NP = 51200
RPT = NP // 16
H = 32
F = 64
E = 800000
EP = 802816
GROUPS = EP // 128
GPT = GROUPS // 16
SCH = 2
NIT = GPT // SCH
DEPTH = 2

_f32 = jnp.float32


@functools.cache
def _mesh():
    return plsc.VectorSubcoreMesh(core_axis_name="c", subcore_axis_name="s")


def _edge_pass_body(x2, k2, src2, dst2, flag16, aggL, aggR,
                    agg_s, xg0, xg1, kg0, kg1, sv0, sv1, dv0, dv1, fv,
                    isem_s0, isem_s1, isem_d0, isem_d1,
                    gsem0, gsem1, ssem0, ssem1):
    c = lax.axis_index("c")
    s = lax.axis_index("s")
    zeros = jnp.zeros((16,), _f32)
    ones = jnp.ones((16,), _f32)
    xgs = (xg0, xg1)
    kgs = (kg0, kg1)
    svs = (sv0, sv1)
    dvs = (dv0, dv1)
    gsems = (gsem0, gsem1)
    ssems = (ssem0, ssem1)
    isems_s = (isem_s0, isem_s1)
    isems_d = (isem_d0, isem_d1)

    pltpu.sync_copy(flag16, fv)
    is_deg = jnp.max(fv[...]) > 0
    not_deg = jnp.logical_not(is_deg)

    @pl.loop(0, 128, unroll=8)
    def _(r):
        xg0[r, pl.ds(0, 16)] = zeros
        xg0[r, pl.ds(16, 16)] = zeros

    for j in range(RPT // 128):
        pltpu.sync_copy(xg0, agg_s.at[pl.ds(s * RPT + j * 128, 128)])
    plsc.subcore_barrier()

    @pl.when(is_deg)
    def _():
        @pl.loop(0, 128, unroll=8)
        def _(r):
            xg0[r, pl.ds(0, 16)] = ones
            xg0[r, pl.ds(16, 16)] = ones
            xg1[r, pl.ds(0, 16)] = ones
            xg1[r, pl.ds(16, 16)] = ones

    g0 = s * GPT
    NCH = GPT

    def issue_gk(ch, p):
        pltpu.async_copy(x2.at[c].at[svs[p].at[0]], xgs[p], gsems[p])
        pltpu.async_copy(k2.at[c].at[pl.ds((g0 + ch) * 128, 128)],
                         kgs[p], gsems[p])

    def wait_gk(ch, p):
        pltpu.make_async_copy(x2.at[c].at[svs[p].at[0]], xgs[p], gsems[p]).wait()
        pltpu.make_async_copy(k2.at[c].at[pl.ds((g0 + ch) * 128, 128)],
                              kgs[p], gsems[p]).wait()

    @pl.when(not_deg)
    def _():
        pltpu.sync_copy(src2.at[pl.ds(g0, 1)], sv0)
        issue_gk(0, 0)
        pltpu.async_copy(src2.at[pl.ds(g0 + 1, 1)], sv1, isem_s1)
    pltpu.async_copy(dst2.at[pl.ds(g0, 1)], dv0, isem_d0)

    @pl.loop(0, NCH // 2)
    def _(j):
        for phase in (0, 1):
            ch = 2 * j + phase
            p = phase
            q = 1 - phase

            @pl.when(ch >= 1)
            def _():
                pltpu.make_async_copy(xgs[q], agg_s.at[dvs[q].at[0]],
                                      ssems[q]).wait()

            @pl.when(ch + 1 < NCH)
            def _():
                pltpu.async_copy(dst2.at[pl.ds(g0 + ch + 1, 1)], dvs[q],
                                 isems_d[q])

            @pl.when(not_deg)
            def _():
                @pl.when(ch + 1 < NCH)
                def _():
                    pltpu.make_async_copy(src2.at[pl.ds(g0 + ch + 1, 1)],
                                          svs[q], isems_s[q]).wait()
                    issue_gk(ch + 1, q)

                wait_gk(ch, p)

                @pl.when(ch + 2 < NCH)
                def _():
                    pltpu.async_copy(src2.at[pl.ds(g0 + ch + 2, 1)],
                                     svs[p], isems_s[p])

                @pl.loop(0, 128, unroll=8)
                def _(r):
                    xgs[p][r, pl.ds(0, 16)] = (xgs[p][r, pl.ds(0, 16)]
                                               * kgs[p][r, pl.ds(0, 16)])
                    xgs[p][r, pl.ds(16, 16)] = (xgs[p][r, pl.ds(16, 16)]
                                                * kgs[p][r, pl.ds(16, 16)])

            pltpu.make_async_copy(dst2.at[pl.ds(g0 + ch, 1)], dvs[p],
                                  isems_d[p]).wait()
            pltpu.async_copy(xgs[p], agg_s.at[dvs[p].at[0]], ssems[p],
                             add=True)

    pltpu.make_async_copy(xg1, agg_s.at[dv1.at[0]], ssem1).wait()

    plsc.subcore_barrier()

    @pl.when(c == 0)
    def _():
        pltpu.sync_copy(agg_s.at[pl.ds(s * RPT, RPT)], aggL.at[pl.ds(s * RPT, RPT)])

    @pl.when(c == 1)
    def _():
        pltpu.sync_copy(agg_s.at[pl.ds(s * RPT, RPT)], aggR.at[pl.ds(s * RPT, RPT)])


@functools.cache
def _edge_pass_fn():
    return pl.kernel(
        _edge_pass_body,
        out_type=[jax.ShapeDtypeStruct((NP, H), _f32),
                  jax.ShapeDtypeStruct((NP, H), _f32)],
        mesh=_mesh(),
        compiler_params=pltpu.CompilerParams(use_tc_tiling_on_sc=False,
                                            needs_layout_passes=False),
        scratch_types=[
            pltpu.VMEM_SHARED((NP, H), _f32),
            pltpu.VMEM((128, H), _f32),
            pltpu.VMEM((128, H), _f32),
            pltpu.VMEM((128, H), _f32),
            pltpu.VMEM((128, H), _f32),
            pltpu.VMEM((1, 128), jnp.int32),
            pltpu.VMEM((1, 128), jnp.int32),
            pltpu.VMEM((1, 128), jnp.int32),
            pltpu.VMEM((1, 128), jnp.int32),
            pltpu.VMEM((16,), jnp.int32),
            pltpu.SemaphoreType.DMA,
            pltpu.SemaphoreType.DMA,
            pltpu.SemaphoreType.DMA,
            pltpu.SemaphoreType.DMA,
            pltpu.SemaphoreType.DMA,
            pltpu.SemaphoreType.DMA,
            pltpu.SemaphoreType.DMA,
            pltpu.SemaphoreType.DMA,
        ],
    )


def _edge_pass(x2, k2, src2, dst2, deg_mode=False):
    flag = jnp.full((16,), 1 if deg_mode else 0, jnp.int32)
    return _edge_pass_fn()(x2, k2, src2, dst2, flag)


_NB_TC = 2000
_NGRID = N // _NB_TC
_EB = 8192
_EGRID = EP // _EB


def _encoder_body(xin_ref, w1_ref, b1_ref, w2_ref, b2_ref, o_ref):
    h = jax.nn.gelu(jnp.dot(xin_ref[...], w1_ref[...],
                            preferred_element_type=_f32) + b1_ref[...])
    x = jax.nn.gelu(jnp.dot(h, w2_ref[...],
                            preferred_element_type=_f32) + b2_ref[...])
    o_ref[0] = x[:, :H]
    o_ref[1] = x[:, H:]


def _encoder(x_in, Wp1, bp1, Wp2, bp2):
    return pl.pallas_call(
        _encoder_body,
        grid=(_NGRID,),
        in_specs=[
            pl.BlockSpec((_NB_TC, 12), lambda i: (i, 0)),
            pl.BlockSpec((12, H), lambda i: (0, 0)),
            pl.BlockSpec((H,), lambda i: (0,)),
            pl.BlockSpec((H, F), lambda i: (0, 0)),
            pl.BlockSpec((F,), lambda i: (0,)),
        ],
        out_specs=pl.BlockSpec((2, _NB_TC, H), lambda i: (0, i, 0)),
        out_shape=jax.ShapeDtypeStruct((2, NP, H), _f32),
    )(x_in, Wp1, bp1, Wp2, bp2)


def _edge_mlp_body(ea_ref, w1_ref, b1_ref, w2_ref, b2_ref, k0_ref, k1_ref):
    e = ea_ref[...]
    for l, out in ((0, k0_ref), (1, k1_ref)):
        h = jax.nn.gelu(jnp.dot(e, w1_ref[l], preferred_element_type=_f32)
                        + b1_ref[l])
        k = jnp.dot(h, w2_ref[l], preferred_element_type=_f32) + b2_ref[l]
        out[0] = k[:, :H]
        out[1] = k[:, H:]


def _edge_mlp(ea_p, Wk1i, bk1i, Wk2i, bk2i):
    return pl.pallas_call(
        _edge_mlp_body,
        grid=(_EGRID,),
        in_specs=[
            pl.BlockSpec((_EB, 8), lambda i: (i, 0)),
            pl.BlockSpec((2, 8, F), lambda i: (0, 0, 0)),
            pl.BlockSpec((2, F), lambda i: (0, 0)),
            pl.BlockSpec((2, F, F), lambda i: (0, 0, 0)),
            pl.BlockSpec((2, F), lambda i: (0, 0)),
        ],
        out_specs=[pl.BlockSpec((2, _EB, H), lambda i: (0, i, 0)),
                   pl.BlockSpec((2, _EB, H), lambda i: (0, i, 0))],
        out_shape=[jax.ShapeDtypeStruct((2, EP, H), _f32),
                   jax.ShapeDtypeStruct((2, EP, H), _f32)],
    )(ea_p, Wk1i, bk1i, Wk2i, bk2i)


def _node_update_body(x2_ref, aL_ref, aR_ref, deg_ref, wr_ref, br_ref, o_ref):
    x = jnp.concatenate([x2_ref[0], x2_ref[1]], axis=-1)
    agg = jnp.concatenate([aL_ref[...], aR_ref[...]], axis=-1)
    rdeg = 1.0 / jnp.maximum(deg_ref[...][:, 0:1], 1.0)
    y = jax.nn.gelu(jnp.dot(x, wr_ref[...], preferred_element_type=_f32)
                    + br_ref[...] + agg * rdeg)
    o_ref[0] = y[:, :H]
    o_ref[1] = y[:, H:]


def _node_update(x2, aggL, aggR, deg, Wr_il, br_il):
    return pl.pallas_call(
        _node_update_body,
        grid=(_NGRID,),
        in_specs=[
            pl.BlockSpec((2, _NB_TC, H), lambda i: (0, i, 0)),
            pl.BlockSpec((_NB_TC, H), lambda i: (i, 0)),
            pl.BlockSpec((_NB_TC, H), lambda i: (i, 0)),
            pl.BlockSpec((_NB_TC, H), lambda i: (i, 0)),
            pl.BlockSpec((F, F), lambda i: (0, 0)),
            pl.BlockSpec((F,), lambda i: (0,)),
        ],
        out_specs=pl.BlockSpec((2, _NB_TC, H), lambda i: (0, i, 0)),
        out_shape=jax.ShapeDtypeStruct((2, NP, H), _f32),
    )(x2, aggL, aggR, deg, Wr_il, br_il)


def _block_out_body(x2_ref, w_ref, b_ref, o_ref):
    x = jnp.concatenate([x2_ref[0], x2_ref[1]], axis=-1)
    y = jnp.dot(x, w_ref[...], preferred_element_type=_f32) + b_ref[...]
    o_ref[0] = y[:, :H]
    o_ref[1] = y[:, H:]


def _block_out_res_body(x2_ref, w_ref, b_ref, r2_ref, o_ref):
    x = jnp.concatenate([x2_ref[0], x2_ref[1]], axis=-1)
    y = jnp.dot(x, w_ref[...], preferred_element_type=_f32) + b_ref[...]
    o_ref[0] = y[:, :H] + r2_ref[0]
    o_ref[1] = y[:, H:] + r2_ref[1]


def _block_out(x2, Wout_i, bout_i, res2=None):
    spec2 = pl.BlockSpec((2, _NB_TC, H), lambda i: (0, i, 0))
    in_specs = [spec2,
                pl.BlockSpec((F, F), lambda i: (0, 0)),
                pl.BlockSpec((F,), lambda i: (0,))]
    args = [x2, Wout_i, bout_i]
    body = _block_out_body
    if res2 is not None:
        in_specs.append(spec2)
        args.append(res2)
        body = _block_out_res_body
    return pl.pallas_call(
        body,
        grid=(_NGRID,),
        in_specs=in_specs,
        out_specs=spec2,
        out_shape=jax.ShapeDtypeStruct((2, NP, H), _f32),
    )(*args)


def _decoder_body(a2_ref, b2_ref, w1_ref, b1_ref, w2_ref, b2b_ref, o_ref):
    x = jnp.concatenate([a2_ref[0] + b2_ref[0], a2_ref[1] + b2_ref[1]], axis=-1)
    h = jax.nn.gelu(jnp.dot(x, w1_ref[...], preferred_element_type=_f32)
                    + b1_ref[...])
    o_ref[...] = jnp.dot(h, w2_ref[...], preferred_element_type=_f32) + b2b_ref[...]


def _decoder(a2, b2, Wd1, bd1, Wd2, bd2):
    spec2 = pl.BlockSpec((2, _NB_TC, H), lambda i: (0, i, 0))
    return pl.pallas_call(
        _decoder_body,
        grid=(_NGRID,),
        in_specs=[
            spec2, spec2,
            pl.BlockSpec((F, H), lambda i: (0, 0)),
            pl.BlockSpec((H,), lambda i: (0,)),
            pl.BlockSpec((H, 1), lambda i: (0, 0)),
            pl.BlockSpec((1,), lambda i: (0,)),
        ],
        out_specs=pl.BlockSpec((_NB_TC, 1), lambda i: (i, 0)),
        out_shape=jax.ShapeDtypeStruct((N, 1), _f32),
    )(a2, b2, Wd1, bd1, Wd2, bd2)


def _prep_edges(ei, ea):
    src = jnp.concatenate([ei[0], jnp.zeros((EP - E,), ei.dtype)])
    dst = jnp.concatenate([ei[1], jnp.full((EP - E,), N, ei.dtype)])
    ea_p = jnp.pad(ea, ((0, EP - E), (0, 3)))
    return (src.astype(jnp.int32).reshape(GROUPS, 128),
            dst.astype(jnp.int32).reshape(GROUPS, 128), ea_p)


def kernel(nodes, grid, edge_index_11, edge_index_12, edge_index_22, edge_index_23, edge_index_33, edge_index_32, edge_index_21, edge_attr_11, edge_attr_12, edge_attr_22, edge_attr_23, edge_attr_33, edge_attr_32, edge_attr_21, batch_size, image_size, Wp1, bp1, Wp2, bp2, Wd1, bd1, Wd2, bd2, Wk1, bk1, Wk2, bk2, Wr, br, Wout, bout):
    eis = [edge_index_11, edge_index_12, edge_index_22, edge_index_23,
           edge_index_33, edge_index_32, edge_index_21]
    eas = [edge_attr_11, edge_attr_12, edge_attr_22, edge_attr_23,
           edge_attr_33, edge_attr_32, edge_attr_21]
    prepped = [_prep_edges(ei, ea) for ei, ea in zip(eis, eas)]
    srcs = [p[0] for p in prepped]
    dsts = [p[1] for p in prepped]
    eaps = [p[2] for p in prepped]

    Wk1p = jnp.pad(Wk1, ((0, 0), (0, 0), (0, 3), (0, 0)))

    x_in = jnp.concatenate([nodes, grid], axis=-1)
    x2 = _encoder(x_in, Wp1, bp1, Wp2, bp2)

    deg_cache = {}

    def gno(x2_in, b_idx, i, res2=None):
        k0, k1 = _edge_mlp(eaps[b_idx], Wk1p[i], bk1[i], Wk2[i], bk2[i])
        if b_idx not in deg_cache:
            deg_cache[b_idx], _ = _edge_pass(x2_in, k0, srcs[b_idx],
                                             dsts[b_idx], deg_mode=True)
        deg = deg_cache[b_idx]
        h2 = x2_in
        for l, k2 in ((0, k0), (1, k1)):
            aggL, aggR = _edge_pass(h2, k2, srcs[b_idx], dsts[b_idx])
            h2 = _node_update(h2, aggL, aggR, deg, Wr[i, l], br[i, l])
        return _block_out(h2, Wout[i], bout[i], res2)

    n11 = gno(x2, 0, 0)
    n12 = gno(x2, 1, 3)
    n22 = gno(n12, 2, 1)
    n23 = gno(n12, 3, 4)
    n33 = gno(n23, 4, 2)
    n32p = gno(n33, 5, 5, res2=n22)
    n21 = gno(n32p, 6, 6)

    return _decoder(n21, n11, Wd1, bd1, Wd2, bd2)


# speedup vs baseline: 2.9509x; 1.0434x over previous
import functools

import jax
import jax.numpy as jnp
from jax import lax
from jax.experimental import pallas as pl
from jax.experimental.pallas import tpu as pltpu
from jax.experimental.pallas import tpu_sc as plsc

N = 50000
NP = 50048
RPT = NP // 16
H = 32
F = 64
E = 800000
EP = 802816
GROUPS = EP // 128
GPT = GROUPS // 16
SCH = 2
NIT = GPT // SCH
DEPTH = 2

_f32 = jnp.float32


@functools.cache
def _mesh():
    return plsc.VectorSubcoreMesh(core_axis_name="c", subcore_axis_name="s")


def _edge_pass_body(x2, k2, src2, dst2, flag16, aggL, aggR,
                    agg_s, xg0, xg1, xg2, xg3, kg0, kg1, kg2, kg3,
                    sv0, sv1, sv2, sv3, dv0, dv1, dv2, dv3, fv,
                    isem_s0, isem_s1, isem_s2, isem_s3,
                    isem_d0, isem_d1, isem_d2, isem_d3,
                    gsem0, gsem1, gsem2, gsem3,
                    ssem0, ssem1, ssem2, ssem3):
    c = lax.axis_index("c")
    s = lax.axis_index("s")
    zeros = jnp.zeros((16,), _f32)
    ones = jnp.ones((16,), _f32)
    xgs = (xg0, xg1, xg2, xg3)
    kgs = (kg0, kg1, kg2, kg3)
    svs = (sv0, sv1, sv2, sv3)
    dvs = (dv0, dv1, dv2, dv3)
    isems_s = (isem_s0, isem_s1, isem_s2, isem_s3)
    isems_d = (isem_d0, isem_d1, isem_d2, isem_d3)
    gsems = (gsem0, gsem1, gsem2, gsem3)
    ssems = (ssem0, ssem1, ssem2, ssem3)

    pltpu.sync_copy(flag16, fv)
    is_deg = jnp.max(fv[...]) > 0
    not_deg = jnp.logical_not(is_deg)

    @pl.loop(0, 128, unroll=8)
    def _(r):
        xg0[r, pl.ds(0, 16)] = zeros
        xg0[r, pl.ds(16, 16)] = zeros

    for j in range(RPT // 128):
        pltpu.sync_copy(xg0, agg_s.at[pl.ds(s * RPT + j * 128, 128)])
    if RPT % 128:
        pltpu.sync_copy(xg0.at[pl.ds(0, RPT % 128)],
                        agg_s.at[pl.ds(s * RPT + (RPT // 128) * 128, RPT % 128)])
    plsc.subcore_barrier()

    @pl.when(is_deg)
    def _():
        @pl.loop(0, 128, unroll=8)
        def _(r):
            for xgb in xgs:
                xgb[r, pl.ds(0, 16)] = ones
                xgb[r, pl.ds(16, 16)] = ones

    g0 = s * GPT
    NCH = GPT

    def issue_gk(ch, b):
        pltpu.async_copy(x2.at[c].at[svs[b].at[0]], xgs[b], gsems[b])
        pltpu.async_copy(k2.at[c].at[pl.ds((g0 + ch) * 128, 128)],
                         kgs[b], gsems[b])

    def wait_gk(ch, b):
        pltpu.make_async_copy(x2.at[c].at[svs[b].at[0]], xgs[b], gsems[b]).wait()
        pltpu.make_async_copy(k2.at[c].at[pl.ds((g0 + ch) * 128, 128)],
                              kgs[b], gsems[b]).wait()

    @pl.when(not_deg)
    def _():
        pltpu.sync_copy(src2.at[pl.ds(g0, 1)], sv0)
        issue_gk(0, 0)
        pltpu.sync_copy(src2.at[pl.ds(g0 + 1, 1)], sv1)
        issue_gk(1, 1)
        pltpu.async_copy(src2.at[pl.ds(g0 + 2, 1)], sv2, isem_s2)
    pltpu.async_copy(dst2.at[pl.ds(g0, 1)], dv0, isem_d0)
    pltpu.async_copy(dst2.at[pl.ds(g0 + 1, 1)], dv1, isem_d1)

    @pl.loop(0, NCH // 4)
    def _(j):
        for phase in (0, 1, 2, 3):
            ch = 4 * j + phase
            b = phase
            bm1 = (phase - 1) % 4
            b2 = (phase + 2) % 4
            b3 = (phase + 3) % 4

            @pl.when(ch >= 1)
            def _():
                pltpu.make_async_copy(xgs[bm1], agg_s.at[dvs[bm1].at[0]],
                                      ssems[bm1]).wait()

            @pl.when(ch + 2 < NCH)
            def _():
                pltpu.async_copy(dst2.at[pl.ds(g0 + ch + 2, 1)], dvs[b2],
                                 isems_d[b2])

            @pl.when(not_deg)
            def _():
                @pl.when(ch + 2 < NCH)
                def _():
                    pltpu.make_async_copy(src2.at[pl.ds(g0 + ch + 2, 1)],
                                          svs[b2], isems_s[b2]).wait()
                    issue_gk(ch + 2, b2)

                @pl.when(ch + 3 < NCH)
                def _():
                    pltpu.async_copy(src2.at[pl.ds(g0 + ch + 3, 1)],
                                     svs[b3], isems_s[b3])

                wait_gk(ch, b)

                @pl.loop(0, 128, unroll=8)
                def _(r):
                    kv = kgs[b][r, pl.ds(0, 32)]
                    ka, kb = plsc.unpack(kv, format=plsc.PackFormat.INTERLEAVED,
                                         preferred_element_type=_f32)
                    xgs[b][r, pl.ds(0, 16)] = xgs[b][r, pl.ds(0, 16)] * ka
                    xgs[b][r, pl.ds(16, 16)] = xgs[b][r, pl.ds(16, 16)] * kb

            pltpu.make_async_copy(dst2.at[pl.ds(g0 + ch, 1)], dvs[b],
                                  isems_d[b]).wait()
            pltpu.async_copy(xgs[b], agg_s.at[dvs[b].at[0]], ssems[b],
                             add=True)

    pltpu.make_async_copy(xg3, agg_s.at[dv3.at[0]], ssem3).wait()

    plsc.subcore_barrier()

    @pl.when(c == 0)
    def _():
        pltpu.sync_copy(agg_s.at[pl.ds(s * RPT, RPT)], aggL.at[pl.ds(s * RPT, RPT)])

    @pl.when(c == 1)
    def _():
        pltpu.sync_copy(agg_s.at[pl.ds(s * RPT, RPT)], aggR.at[pl.ds(s * RPT, RPT)])


@functools.cache
def _edge_pass_fn():
    return pl.kernel(
        _edge_pass_body,
        out_type=[jax.ShapeDtypeStruct((NP, H), _f32),
                  jax.ShapeDtypeStruct((NP, H), _f32)],
        mesh=_mesh(),
        compiler_params=pltpu.CompilerParams(use_tc_tiling_on_sc=False,
                                            needs_layout_passes=False),
        scratch_types=(
            [pltpu.VMEM_SHARED((NP, H), _f32)]
            + [pltpu.VMEM((128, H), _f32) for _ in range(4)]
            + [pltpu.VMEM((128, H), jnp.bfloat16) for _ in range(4)]
            + [pltpu.VMEM((1, 128), jnp.int32) for _ in range(8)]
            + [pltpu.VMEM((16,), jnp.int32)]
            + [pltpu.SemaphoreType.DMA for _ in range(16)]
        ),
    )


def _edge_pass(x2, k2, src2, dst2, deg_mode=False):
    flag = jnp.full((16,), 1 if deg_mode else 0, jnp.int32)
    return _edge_pass_fn()(x2, k2, src2, dst2, flag)


_NB_TC = 2000
_NGRID = N // _NB_TC
_EB = 4096
_EGRID = EP // _EB


def _encoder_body(xin_ref, w1_ref, b1_ref, w2_ref, b2_ref, o_ref):
    h = jax.nn.gelu(jnp.dot(xin_ref[...], w1_ref[...],
                            preferred_element_type=_f32) + b1_ref[...])
    x = jax.nn.gelu(jnp.dot(h, w2_ref[...],
                            preferred_element_type=_f32) + b2_ref[...])
    o_ref[0] = x[:, :H]
    o_ref[1] = x[:, H:]


def _encoder(x_in, Wp1, bp1, Wp2, bp2):
    return pl.pallas_call(
        _encoder_body,
        grid=(_NGRID,),
        in_specs=[
            pl.BlockSpec((_NB_TC, 12), lambda i: (i, 0)),
            pl.BlockSpec((12, H), lambda i: (0, 0)),
            pl.BlockSpec((H,), lambda i: (0,)),
            pl.BlockSpec((H, F), lambda i: (0, 0)),
            pl.BlockSpec((F,), lambda i: (0,)),
        ],
        out_specs=pl.BlockSpec((2, _NB_TC, H), lambda i: (0, i, 0)),
        out_shape=jax.ShapeDtypeStruct((2, NP, H), _f32),
    )(x_in, Wp1, bp1, Wp2, bp2)


def _edge_mlp_body(ea_ref, w1_ref, b1_ref, w2_ref, b2_ref, k0_ref, k1_ref):
    e = ea_ref[...]
    for l, out in ((0, k0_ref), (1, k1_ref)):
        h = jax.nn.gelu(jnp.dot(e, w1_ref[l], preferred_element_type=_f32)
                        + b1_ref[l])
        k = jnp.dot(h, w2_ref[l], preferred_element_type=_f32) + b2_ref[l]
        kb = k.astype(jnp.bfloat16)
        out[0] = kb[:, :H]
        out[1] = kb[:, H:]


def _edge_mlp(ea_p, Wk1i, bk1i, Wk2i, bk2i):
    return pl.pallas_call(
        _edge_mlp_body,
        grid=(_EGRID,),
        in_specs=[
            pl.BlockSpec((_EB, 8), lambda i: (i, 0)),
            pl.BlockSpec((2, 8, F), lambda i: (0, 0, 0)),
            pl.BlockSpec((2, F), lambda i: (0, 0)),
            pl.BlockSpec((2, F, F), lambda i: (0, 0, 0)),
            pl.BlockSpec((2, F), lambda i: (0, 0)),
        ],
        out_specs=[pl.BlockSpec((2, _EB, H), lambda i: (0, i, 0)),
                   pl.BlockSpec((2, _EB, H), lambda i: (0, i, 0))],
        out_shape=[jax.ShapeDtypeStruct((2, EP, H), jnp.bfloat16),
                   jax.ShapeDtypeStruct((2, EP, H), jnp.bfloat16)],
    )(ea_p, Wk1i, bk1i, Wk2i, bk2i)


def _node_update_body(x2_ref, aL_ref, aR_ref, deg_ref, wr_ref, br_ref, o_ref):
    x = jnp.concatenate([x2_ref[0], x2_ref[1]], axis=-1)
    agg = jnp.concatenate([aL_ref[...], aR_ref[...]], axis=-1)
    rdeg = 1.0 / jnp.maximum(deg_ref[...][:, 0:1], 1.0)
    y = jax.nn.gelu(jnp.dot(x, wr_ref[...], preferred_element_type=_f32)
                    + br_ref[...] + agg * rdeg)
    o_ref[0] = y[:, :H]
    o_ref[1] = y[:, H:]


def _node_update(x2, aggL, aggR, deg, Wr_il, br_il):
    return pl.pallas_call(
        _node_update_body,
        grid=(_NGRID,),
        in_specs=[
            pl.BlockSpec((2, _NB_TC, H), lambda i: (0, i, 0)),
            pl.BlockSpec((_NB_TC, H), lambda i: (i, 0)),
            pl.BlockSpec((_NB_TC, H), lambda i: (i, 0)),
            pl.BlockSpec((_NB_TC, H), lambda i: (i, 0)),
            pl.BlockSpec((F, F), lambda i: (0, 0)),
            pl.BlockSpec((F,), lambda i: (0,)),
        ],
        out_specs=pl.BlockSpec((2, _NB_TC, H), lambda i: (0, i, 0)),
        out_shape=jax.ShapeDtypeStruct((2, NP, H), _f32),
    )(x2, aggL, aggR, deg, Wr_il, br_il)


def _block_out_body(x2_ref, w_ref, b_ref, o_ref):
    x = jnp.concatenate([x2_ref[0], x2_ref[1]], axis=-1)
    y = jnp.dot(x, w_ref[...], preferred_element_type=_f32) + b_ref[...]
    o_ref[0] = y[:, :H]
    o_ref[1] = y[:, H:]


def _block_out_res_body(x2_ref, w_ref, b_ref, r2_ref, o_ref):
    x = jnp.concatenate([x2_ref[0], x2_ref[1]], axis=-1)
    y = jnp.dot(x, w_ref[...], preferred_element_type=_f32) + b_ref[...]
    o_ref[0] = y[:, :H] + r2_ref[0]
    o_ref[1] = y[:, H:] + r2_ref[1]


def _block_out(x2, Wout_i, bout_i, res2=None):
    spec2 = pl.BlockSpec((2, _NB_TC, H), lambda i: (0, i, 0))
    in_specs = [spec2,
                pl.BlockSpec((F, F), lambda i: (0, 0)),
                pl.BlockSpec((F,), lambda i: (0,))]
    args = [x2, Wout_i, bout_i]
    body = _block_out_body
    if res2 is not None:
        in_specs.append(spec2)
        args.append(res2)
        body = _block_out_res_body
    return pl.pallas_call(
        body,
        grid=(_NGRID,),
        in_specs=in_specs,
        out_specs=spec2,
        out_shape=jax.ShapeDtypeStruct((2, NP, H), _f32),
    )(*args)


def _decoder_body(a2_ref, b2_ref, w1_ref, b1_ref, w2_ref, b2b_ref, o_ref):
    x = jnp.concatenate([a2_ref[0] + b2_ref[0], a2_ref[1] + b2_ref[1]], axis=-1)
    h = jax.nn.gelu(jnp.dot(x, w1_ref[...], preferred_element_type=_f32)
                    + b1_ref[...])
    o_ref[...] = jnp.dot(h, w2_ref[...], preferred_element_type=_f32) + b2b_ref[...]


def _decoder(a2, b2, Wd1, bd1, Wd2, bd2):
    spec2 = pl.BlockSpec((2, _NB_TC, H), lambda i: (0, i, 0))
    return pl.pallas_call(
        _decoder_body,
        grid=(_NGRID,),
        in_specs=[
            spec2, spec2,
            pl.BlockSpec((F, H), lambda i: (0, 0)),
            pl.BlockSpec((H,), lambda i: (0,)),
            pl.BlockSpec((H, 1), lambda i: (0, 0)),
            pl.BlockSpec((1,), lambda i: (0,)),
        ],
        out_specs=pl.BlockSpec((_NB_TC, 1), lambda i: (i, 0)),
        out_shape=jax.ShapeDtypeStruct((N, 1), _f32),
    )(a2, b2, Wd1, bd1, Wd2, bd2)


def _prep_edges(ei, ea):
    src = jnp.concatenate([ei[0], jnp.zeros((EP - E,), ei.dtype)])
    dst = jnp.concatenate([ei[1], jnp.full((EP - E,), N, ei.dtype)])
    ea_p = jnp.pad(ea, ((0, EP - E), (0, 3)))
    return (src.astype(jnp.int32).reshape(GROUPS, 128),
            dst.astype(jnp.int32).reshape(GROUPS, 128), ea_p)


def kernel(nodes, grid, edge_index_11, edge_index_12, edge_index_22, edge_index_23, edge_index_33, edge_index_32, edge_index_21, edge_attr_11, edge_attr_12, edge_attr_22, edge_attr_23, edge_attr_33, edge_attr_32, edge_attr_21, batch_size, image_size, Wp1, bp1, Wp2, bp2, Wd1, bd1, Wd2, bd2, Wk1, bk1, Wk2, bk2, Wr, br, Wout, bout):
    eis = [edge_index_11, edge_index_12, edge_index_22, edge_index_23,
           edge_index_33, edge_index_32, edge_index_21]
    eas = [edge_attr_11, edge_attr_12, edge_attr_22, edge_attr_23,
           edge_attr_33, edge_attr_32, edge_attr_21]
    prepped = [_prep_edges(ei, ea) for ei, ea in zip(eis, eas)]
    srcs = [p[0] for p in prepped]
    dsts = [p[1] for p in prepped]
    eaps = [p[2] for p in prepped]

    Wk1p = jnp.pad(Wk1, ((0, 0), (0, 0), (0, 3), (0, 0)))
    perm = [c * H + (j // 2) + 16 * (j % 2) for c in range(2) for j in range(H)]
    kperm = jnp.array(perm, jnp.int32)
    Wk2p = Wk2[:, :, :, kperm]
    bk2p = bk2[:, :, kperm]

    x_in = jnp.concatenate([nodes, grid], axis=-1)
    x2 = _encoder(x_in, Wp1, bp1, Wp2, bp2)

    deg_cache = {}

    def gno(x2_in, b_idx, i, res2=None):
        k0, k1 = _edge_mlp(eaps[b_idx], Wk1p[i], bk1[i], Wk2p[i], bk2p[i])
        if b_idx not in deg_cache:
            deg_cache[b_idx], _ = _edge_pass(x2_in, k0, srcs[b_idx],
                                             dsts[b_idx], deg_mode=True)
        deg = deg_cache[b_idx]
        h2 = x2_in
        for l, k2 in ((0, k0), (1, k1)):
            aggL, aggR = _edge_pass(h2, k2, srcs[b_idx], dsts[b_idx])
            h2 = _node_update(h2, aggL, aggR, deg, Wr[i, l], br[i, l])
        return _block_out(h2, Wout[i], bout[i], res2)

    n11 = gno(x2, 0, 0)
    n12 = gno(x2, 1, 3)
    n22 = gno(n12, 2, 1)
    n23 = gno(n12, 3, 4)
    n33 = gno(n23, 4, 2)
    n32p = gno(n33, 5, 5, res2=n22)
    n21 = gno(n32p, 6, 6)

    return _decoder(n21, n11, Wd1, bd1, Wd2, bd2)
